# baseline (speedup 1.0000x reference)
import functools

import jax
import jax.numpy as jnp
from jax import lax
from jax.experimental import pallas as pl
from jax.experimental.pallas import tpu as pltpu
from jax.experimental.pallas import tpu_sc as plsc

_BATCH = 16384
_EMB = 32
_NC = 2
_NS = 16
_NW = _NC * _NS
_BPW = _BATCH // _NW

_BT = 2048


def _sc_gather(user_idx, item_idx, user_table, item_table):
    mesh = plsc.VectorSubcoreMesh(core_axis_name="c", subcore_axis_name="s")

    @functools.partial(
        pl.kernel,
        out_type=(
            jax.ShapeDtypeStruct((_BATCH, _EMB), jnp.float32),
            jax.ShapeDtypeStruct((_BATCH, _EMB), jnp.float32),
        ),
        mesh=mesh,
        compiler_params=pltpu.CompilerParams(use_tc_tiling_on_sc=False),
        scratch_types=[
            pltpu.VMEM((_BPW,), jnp.int32),
            pltpu.VMEM((_BPW,), jnp.int32),
            pltpu.VMEM((_BPW, _EMB), jnp.float32),
            pltpu.VMEM((_BPW, _EMB), jnp.float32),
            pltpu.SemaphoreType.DMA,
            pltpu.SemaphoreType.DMA,
        ],
    )
    def gather(uidx_hbm, iidx_hbm, utab_hbm, itab_hbm, uout_hbm, iout_hbm,
               uidx_v, iidx_v, urows_v, irows_v, usem, isem):
        wid = lax.axis_index("s") * _NC + lax.axis_index("c")
        base = wid * _BPW
        pltpu.sync_copy(uidx_hbm.at[pl.ds(base, _BPW)], uidx_v)
        pltpu.sync_copy(iidx_hbm.at[pl.ds(base, _BPW)], iidx_v)
        cu = pltpu.async_copy(utab_hbm.at[uidx_v], urows_v, usem)
        ci = pltpu.async_copy(itab_hbm.at[iidx_v], irows_v, isem)
        cu.wait()
        ci.wait()
        pltpu.sync_copy(urows_v, uout_hbm.at[pl.ds(base, _BPW)])
        pltpu.sync_copy(irows_v, iout_hbm.at[pl.ds(base, _BPW)])

    return gather(user_idx, item_idx, user_table, item_table)


def _mlp_body(u_ref, i_ref, w1u_ref, w1i_ref, b1_ref, w2_ref, b2_ref,
              w3_ref, b3_ref, o_ref):
    dn = (((1,), (1,)), ((), ()))
    x1 = lax.dot_general(u_ref[...], w1u_ref[...], dn,
                         preferred_element_type=jnp.float32)
    x1 = x1 + lax.dot_general(i_ref[...], w1i_ref[...], dn,
                              preferred_element_type=jnp.float32)
    x1 = jnp.maximum(x1 + b1_ref[...], 0.0)
    x2 = lax.dot_general(x1, w2_ref[...], dn,
                         preferred_element_type=jnp.float32)
    x2 = jnp.maximum(x2 + b2_ref[...], 0.0)
    z = jnp.sum(x2 * w3_ref[...], axis=1, keepdims=True)
    z = z + b3_ref[0]
    o_ref[...] = 1.0 / (1.0 + jnp.exp(-z))


def _tc_mlp(u_emb, i_emb, W1u, W1i, b1, W2, b2, W3, b3):
    grid = (_BATCH // _BT,)
    full = lambda shape: pl.BlockSpec(shape, lambda i: (0, 0))
    return pl.pallas_call(
        _mlp_body,
        grid=grid,
        in_specs=[
            pl.BlockSpec((_BT, _EMB), lambda i: (i, 0)),
            pl.BlockSpec((_BT, _EMB), lambda i: (i, 0)),
            full(W1u.shape),
            full(W1i.shape),
            full(b1.shape),
            full(W2.shape),
            full(b2.shape),
            full(W3.shape),
            pl.BlockSpec(memory_space=pltpu.SMEM),
        ],
        out_specs=pl.BlockSpec((_BT, 1), lambda i: (i, 0)),
        out_shape=jax.ShapeDtypeStruct((_BATCH, 1), jnp.float32),
    )(u_emb, i_emb, W1u, W1i, b1, W2, b2, W3, b3)


def kernel(user_idx, item_idx, user_table, item_table, W1, b1, W2, b2, W3, b3):
    uidx = user_idx.astype(jnp.int32)
    iidx = item_idx.astype(jnp.int32)
    u_emb, i_emb = _sc_gather(uidx, iidx, user_table, item_table)
    W1u = W1[:, :_EMB]
    W1i = W1[:, _EMB:]
    return _tc_mlp(u_emb, i_emb, W1u, W1i,
                   b1.reshape(1, -1), W2, b2.reshape(1, -1),
                   W3, b3)
